# Initial kernel scaffold; baseline (speedup 1.0000x reference)
import jax
import jax.numpy as jnp
from jax.experimental import pallas as pl


def kernel(x, action_sel, edge_index, W1, b1, Wc1, bc1, Wc2, bc2, W2, b2, W5, b5, W6, b6, W8, b8):
    raise NotImplementedError("write your pallas kernel here")


# speedup vs baseline: 6.3785x; 6.3785x over previous
import functools

import jax
import jax.numpy as jnp
from jax import lax
from jax.experimental import pallas as pl
from jax.experimental.pallas import tpu as pltpu
from jax.experimental.pallas import tpu_sc as plsc

N = 10000
E = 320000
HID = 128

NC = 2
NS = 16
NW = NC * NS
CH = 128
NCH = 80
EW = CH * NCH
EPAD = EW * NW
NP = 10240
RPT = NP // NS

BN = 2000
BP = 2048

def _deg_body(dst_hbm, out_hbm, didx, ones_v, zbuf, deg_sh):
    c = lax.axis_index("c")
    s = lax.axis_index("s")
    w = c * NS + s

    def _fill(i, carry):
        ones_v[i, :] = jnp.ones((16,), jnp.float32)
        return carry

    lax.fori_loop(0, CH, _fill, 0)

    def _zv(i, carry):
        zbuf[i, :] = jnp.zeros((16,), jnp.float32)
        return carry

    lax.fori_loop(0, 64, _zv, 0)

    def _zs(i, carry):
        pltpu.sync_copy(zbuf, deg_sh.at[pl.ds(s * RPT + i * 64, 64)])
        return carry

    lax.fori_loop(0, RPT // 64, _zs, 0)
    plsc.subcore_barrier()

    base = w * EW

    def _chunk(i, carry):
        pltpu.sync_copy(dst_hbm.at[pl.ds(base + i * CH, CH)], didx)
        pltpu.sync_copy(ones_v, deg_sh.at[didx], add=True)
        return carry

    lax.fori_loop(0, NCH, _chunk, 0)
    plsc.subcore_barrier()
    pltpu.sync_copy(deg_sh.at[pl.ds(s * RPT, RPT)],
                    out_hbm.at[c].at[pl.ds(s * RPT, RPT)])


def _spmm_body(g_hbm, src_hbm, dst_hbm, out_hbm, sidx, didx, rows, zbuf, acc_sh, sem):
    c = lax.axis_index("c")
    s = lax.axis_index("s")
    w = c * NS + s

    def _zv(i, carry):
        zbuf[i // 8, pl.ds((i % 8) * 16, 16)] = jnp.zeros((16,), jnp.float32)
        return carry

    lax.fori_loop(0, 64 * 8, _zv, 0)

    def _zs(i, carry):
        pltpu.sync_copy(zbuf, acc_sh.at[pl.ds(s * RPT + i * 64, 64)])
        return carry

    lax.fori_loop(0, RPT // 64, _zs, 0)
    plsc.subcore_barrier()

    base = w * EW

    def _chunk(i, carry):
        off = base + i * CH
        pltpu.sync_copy(src_hbm.at[pl.ds(off, CH)], sidx)
        pltpu.sync_copy(dst_hbm.at[pl.ds(off, CH)], didx)
        pltpu.async_copy(g_hbm.at[sidx], rows, sem).wait()
        pltpu.sync_copy(rows, acc_sh.at[didx], add=True)
        return carry

    lax.fori_loop(0, NCH, _chunk, 0)
    plsc.subcore_barrier()
    pltpu.sync_copy(acc_sh.at[pl.ds(s * RPT, RPT)],
                    out_hbm.at[c].at[pl.ds(s * RPT, RPT)])


@functools.cache
def _get_deg_sc():
    mesh = plsc.VectorSubcoreMesh(core_axis_name="c", subcore_axis_name="s",
                                  num_cores=NC, num_subcores=NS)
    return pl.kernel(
        _deg_body,
        out_type=jax.ShapeDtypeStruct((NC, NP, 16), jnp.float32),
        mesh=mesh,
        scratch_types=[
            pltpu.VMEM((CH,), jnp.int32),
            pltpu.VMEM((CH, 16), jnp.float32),
            pltpu.VMEM((64, 16), jnp.float32),
            pltpu.VMEM_SHARED((NP, 16), jnp.float32),
        ],
    )


@functools.cache
def _get_spmm_sc():
    mesh = plsc.VectorSubcoreMesh(core_axis_name="c", subcore_axis_name="s",
                                  num_cores=NC, num_subcores=NS)
    return pl.kernel(
        _spmm_body,
        out_type=jax.ShapeDtypeStruct((NC, NP, HID), jnp.float32),
        mesh=mesh,
        scratch_types=[
            pltpu.VMEM((CH,), jnp.int32),
            pltpu.VMEM((CH,), jnp.int32),
            pltpu.VMEM((CH, HID), jnp.float32),
            pltpu.VMEM((64, HID), jnp.float32),
            pltpu.VMEM_SHARED((NP, HID), jnp.float32),
            pltpu.SemaphoreType.DMA,
        ],
    )


def _dinv_body(degp_ref, dinv_ref):
    deg = jnp.sum(degp_ref[...], axis=(0, 2)) * (1.0 / 16.0) + 1.0
    dinv_ref[...] = lax.rsqrt(deg)


_dinv_tc = pl.pallas_call(
    _dinv_body,
    grid=(NP // BP,),
    in_specs=[pl.BlockSpec((NC, BP, 16), lambda i: (0, i, 0))],
    out_specs=pl.BlockSpec((BP,), lambda i: (i,)),
    out_shape=jax.ShapeDtypeStruct((NP,), jnp.float32),
)


def _k1_body(x_ref, w1_ref, b1_ref, wc1_ref, dinv_ref, x1_ref, g1_ref):
    x1 = jnp.dot(x_ref[...], w1_ref[...], preferred_element_type=jnp.float32)
    x1 = x1 + b1_ref[...]
    x1_ref[...] = x1
    g1_ref[...] = jnp.dot(x1, wc1_ref[...],
                          preferred_element_type=jnp.float32) * dinv_ref[...]


_k1_tc = pl.pallas_call(
    _k1_body,
    grid=(N // BN,),
    in_specs=[
        pl.BlockSpec((BN, HID), lambda i: (i, 0)),
        pl.BlockSpec((HID, HID), lambda i: (0, 0)),
        pl.BlockSpec((1, HID), lambda i: (0, 0)),
        pl.BlockSpec((HID, HID), lambda i: (0, 0)),
        pl.BlockSpec((BN, 1), lambda i: (i, 0)),
    ],
    out_specs=[
        pl.BlockSpec((BN, HID), lambda i: (i, 0)),
        pl.BlockSpec((BN, HID), lambda i: (i, 0)),
    ],
    out_shape=[
        jax.ShapeDtypeStruct((N, HID), jnp.float32),
        jax.ShapeDtypeStruct((N, HID), jnp.float32),
    ],
)


def _k2_body(acc_ref, g_ref, dinv_ref, bc_ref, wc_ref, h_ref, gn_ref):
    tot = acc_ref[0] + acc_ref[1] + g_ref[...]
    h = jnp.maximum(tot * dinv_ref[...] + bc_ref[...], 0.0)
    h_ref[...] = h
    gn_ref[...] = jnp.dot(h, wc_ref[...],
                          preferred_element_type=jnp.float32) * dinv_ref[...]


_k2_tc = pl.pallas_call(
    _k2_body,
    grid=(N // BN,),
    in_specs=[
        pl.BlockSpec((NC, BN, HID), lambda i: (0, i, 0)),
        pl.BlockSpec((BN, HID), lambda i: (i, 0)),
        pl.BlockSpec((BN, 1), lambda i: (i, 0)),
        pl.BlockSpec((1, HID), lambda i: (0, 0)),
        pl.BlockSpec((HID, HID), lambda i: (0, 0)),
    ],
    out_specs=[
        pl.BlockSpec((BN, HID), lambda i: (i, 0)),
        pl.BlockSpec((BN, HID), lambda i: (i, 0)),
    ],
    out_shape=[
        jax.ShapeDtypeStruct((N, HID), jnp.float32),
        jax.ShapeDtypeStruct((N, HID), jnp.float32),
    ],
)


def _k3_body(acc_ref, g_ref, dinv_ref, bc_ref, x1_ref, x2_ref,
             w2a_ref, w2b_ref, w2c_ref, b2_ref, asel_ref, nv_ref, pp_ref):
    tot = acc_ref[0] + acc_ref[1] + g_ref[...]
    x3 = jnp.maximum(tot * dinv_ref[...] + bc_ref[...], 0.0)
    nv = (jnp.dot(x1_ref[...], w2a_ref[...], preferred_element_type=jnp.float32)
          + jnp.dot(x2_ref[...], w2b_ref[...], preferred_element_type=jnp.float32)
          + jnp.dot(x3, w2c_ref[...], preferred_element_type=jnp.float32)
          + b2_ref[...])
    nv_ref[...] = nv
    mask = asel_ref[...] == 0
    part = jnp.sum(jnp.where(mask, nv, 0.0), axis=0, keepdims=True)
    row0 = lax.broadcasted_iota(jnp.int32, (8, HID), 0) == 0
    pp_ref[...] = jnp.where(row0, part, 0.0)


_k3_tc = pl.pallas_call(
    _k3_body,
    grid=(N // BN,),
    in_specs=[
        pl.BlockSpec((NC, BN, HID), lambda i: (0, i, 0)),
        pl.BlockSpec((BN, HID), lambda i: (i, 0)),
        pl.BlockSpec((BN, 1), lambda i: (i, 0)),
        pl.BlockSpec((1, HID), lambda i: (0, 0)),
        pl.BlockSpec((BN, HID), lambda i: (i, 0)),
        pl.BlockSpec((BN, HID), lambda i: (i, 0)),
        pl.BlockSpec((HID, HID), lambda i: (0, 0)),
        pl.BlockSpec((HID, HID), lambda i: (0, 0)),
        pl.BlockSpec((HID, HID), lambda i: (0, 0)),
        pl.BlockSpec((1, HID), lambda i: (0, 0)),
        pl.BlockSpec((BN, 1), lambda i: (i, 0)),
    ],
    out_specs=[
        pl.BlockSpec((BN, HID), lambda i: (i, 0)),
        pl.BlockSpec((8, HID), lambda i: (i, 0)),
    ],
    out_shape=[
        jax.ShapeDtypeStruct((N, HID), jnp.float32),
        jax.ShapeDtypeStruct((8 * N // BN, HID), jnp.float32),
    ],
)


def _k5_body(nv_ref, pp_ref, w6_ref, b6_ref, w5a_ref, w5b_ref, b5_ref,
             w8_ref, b8_ref, q_ref):
    pooled = jnp.sum(pp_ref[...], axis=0, keepdims=True)
    grow = jnp.dot(pooled, w6_ref[...], preferred_element_type=jnp.float32)
    grow = grow + b6_ref[...]
    crow = jnp.dot(jnp.maximum(grow, 0.0), w5a_ref[...],
                   preferred_element_type=jnp.float32) + b5_ref[...]
    h = jnp.dot(jnp.maximum(nv_ref[...], 0.0), w5b_ref[...],
                preferred_element_type=jnp.float32) + crow
    h = jnp.maximum(h, 0.0)
    q_ref[...] = jnp.sum(h * w8_ref[...], axis=1, keepdims=True) + b8_ref[...]


_k5_tc = pl.pallas_call(
    _k5_body,
    grid=(N // BN,),
    in_specs=[
        pl.BlockSpec((BN, HID), lambda i: (i, 0)),
        pl.BlockSpec((8 * N // BN, HID), lambda i: (0, 0)),
        pl.BlockSpec((HID, HID), lambda i: (0, 0)),
        pl.BlockSpec((1, HID), lambda i: (0, 0)),
        pl.BlockSpec((HID, HID), lambda i: (0, 0)),
        pl.BlockSpec((HID, HID), lambda i: (0, 0)),
        pl.BlockSpec((1, HID), lambda i: (0, 0)),
        pl.BlockSpec((1, HID), lambda i: (0, 0)),
        pl.BlockSpec((1, 1), lambda i: (0, 0)),
    ],
    out_specs=pl.BlockSpec((BN, 1), lambda i: (i, 0)),
    out_shape=jax.ShapeDtypeStruct((N, 1), jnp.float32),
)


def kernel(x, action_sel, edge_index, W1, b1, Wc1, bc1, Wc2, bc2,
           W2, b2, W5, b5, W6, b6, W8, b8):
    src = edge_index[0].astype(jnp.int32)
    dst = edge_index[1].astype(jnp.int32)
    pad = EPAD - E
    src_p = jnp.concatenate([src, jnp.zeros((pad,), jnp.int32)])
    dst_p = jnp.concatenate([dst, jnp.full((pad,), N, jnp.int32)])

    deg_parts = _get_deg_sc()(dst_p)
    dinv = _dinv_tc(deg_parts)
    dinv2 = dinv[:N, None]

    xp = jnp.pad(x, ((0, 0), (0, HID - x.shape[1])))
    W1p = jnp.zeros((HID, HID), jnp.float32).at[: x.shape[1]].set(W1)

    x1, g1 = _k1_tc(xp, W1p, b1[None, :], Wc1, dinv2)
    spmm = _get_spmm_sc()
    acc1 = spmm(g1, src_p, dst_p)
    x2, g2 = _k2_tc(acc1, g1, dinv2, bc1[None, :], Wc2)
    acc2 = spmm(g2, src_p, dst_p)
    nv, pp = _k3_tc(acc2, g2, dinv2, bc2[None, :],
                    x1, x2, W2[:HID], W2[HID:2 * HID], W2[2 * HID:],
                    b2[None, :], action_sel[:, None].astype(jnp.int32))
    q = _k5_tc(nv, pp, W6, b6[None, :], W5[:HID], W5[HID:],
               b5[None, :], W8.T, b8[None, :])
    return (q[:, 0], nv)
